# Initial kernel scaffold; baseline (speedup 1.0000x reference)
import jax
import jax.numpy as jnp
from jax.experimental import pallas as pl


def kernel(x, W_sc, b_sc, g_sc, be_sc, W0, b0, g0, be0, W1, b1, g1, be1, W2, b2, g2, be2):
    raise NotImplementedError("write your pallas kernel here")


# speedup vs baseline: 11.5281x; 11.5281x over previous
import functools

import jax
import jax.numpy as jnp
from jax import lax
from jax.experimental import pallas as pl
from jax.experimental.pallas import tpu as pltpu
from jax.experimental.pallas import tpu_sc as plsc

B, N, D, K, C = 16, 1024, 16, 20, 64
EPS = 1e-5
NKTOT = B * N * K
NK_CNT = float(B * N * K)
NSC_CNT = float(B * N)
RN = 256
RM = 512
LANES = 128
NROWS = NKTOT // LANES
NW = 32
CHUNKS_PER_W = NROWS // NW
PTS_PER_ROW = LANES // D


def _knn_body(xrows_ref, xt2_ref, idx_ref):
    b = pl.program_id(0)
    xr = xrows_ref[0]
    xi_p = xr[:, 0:1]
    xi_e = xr[:, 1:2]
    xj_p = xt2_ref[0, 0:1, :]
    xj_e = xt2_ref[0, 1:2, :]
    dx = xj_p - xi_p
    dy = xj_e - xi_e
    s = dx * dx + dy * dy
    jota = lax.broadcasted_iota(jnp.int32, (RN, N), 1)
    big = jnp.float32(3.0e38)
    cols = []
    for _ in range(K):
        m = jnp.min(s, axis=1, keepdims=True)
        cand = jnp.where(s == m, jota, N)
        jk = jnp.min(cand, axis=1, keepdims=True)
        cols.append(jk)
        s = jnp.where(jota == jk, big, s)
    idx_ref[0] = jnp.concatenate(cols, axis=1) + b * N


def _knn_call(x):
    xt2 = jnp.transpose(x[:, :, :2], (0, 2, 1))
    return pl.pallas_call(
        _knn_body,
        grid=(B, N // RN),
        in_specs=[
            pl.BlockSpec((1, RN, D), lambda b, r: (b, r, 0)),
            pl.BlockSpec((1, 2, N), lambda b, r: (b, 0, 0)),
        ],
        out_specs=pl.BlockSpec((1, RN, K), lambda b, r: (b, r, 0)),
        out_shape=jax.ShapeDtypeStruct((B, N, K), jnp.int32),
    )(x, xt2)


def _gather_body(xflat_hbm, idx_hbm, out_hbm, x_tile, idx_v, packed_t, sem):
    del sem
    wid = lax.axis_index("s") * 2 + lax.axis_index("c")
    b = wid // 2
    pltpu.sync_copy(xflat_hbm.at[pl.ds(b * N * D, N * D)], x_tile)

    def chunk(t, carry):
        k = t // 4
        cco = t % 4
        row = k * LANES + wid * 4 + cco
        col0 = wid * 512 + cco * LANES
        pltpu.sync_copy(idx_hbm.at[row], idx_v)
        for c in range(8):
            iv = idx_v[pl.ds(16 * c, 16)] & (N - 1)
            fbase = iv << 4
            for d in range(D):
                v = plsc.load_gather(x_tile, [fbase + d])
                packed_t[d, pl.ds(16 * c, 16)] = v
        pltpu.sync_copy(
            packed_t, out_hbm.at[pl.ds(k * D, D), pl.ds(col0, LANES)])
        return carry

    lax.fori_loop(0, CHUNKS_PER_W, chunk, 0)


def _gather_call(x, idx):
    xflat = x.reshape(B * N * D)
    idx2d = jnp.transpose(idx.reshape(B * N, K)).reshape(NROWS, LANES)
    mesh = plsc.VectorSubcoreMesh(core_axis_name="c", subcore_axis_name="s")
    fn = functools.partial(
        pl.kernel,
        mesh=mesh,
        compiler_params=pltpu.CompilerParams(needs_layout_passes=False),
        out_type=jax.ShapeDtypeStruct((K * D, B * N), jnp.float32),
        scratch_types=[
            pltpu.VMEM((N * D,), jnp.float32),
            pltpu.VMEM((LANES,), jnp.int32),
            pltpu.VMEM((D, LANES), jnp.float32),
            pltpu.SemaphoreType.DMA,
        ],
    )(_gather_body)
    return fn(xflat, idx2d)


def _bn_const(s_ref, q_ref, g_ref, be_ref, cnt):
    mean = s_ref[...] / cnt
    var = q_ref[...] / cnt - mean * mean
    a = g_ref[...] * lax.rsqrt(var + EPS)
    c = be_ref[...] - a * mean
    return a, c


def _dot(a, b):
    return jnp.dot(a, b, preferred_element_type=jnp.float32)


def _h0_list(xc_ref, xk_ref, w0b_ref, w0c_ref, b0_ref):
    hc = _dot(w0c_ref[...], xc_ref[...]) + b0_ref[...]
    out = []
    for k in range(K):
        xkk = xk_ref[k * D:(k + 1) * D, :]
        out.append(_dot(w0b_ref[...], xkk) + hc)
    return out


def _accum(ref, val):
    @pl.when(pl.program_id(0) == 0)
    def _():
        ref[...] = jnp.zeros_like(ref)
    ref[...] += val


def _sumsq_accum(s_ref, q_ref, hs):
    s = jnp.zeros((C, 1), jnp.float32)
    q = jnp.zeros((C, 1), jnp.float32)
    for h in hs:
        s += jnp.sum(h, axis=1, keepdims=True)
        q += jnp.sum(h * h, axis=1, keepdims=True)
    _accum(s_ref, s)
    _accum(q_ref, q)


def _stats0_body(xc_ref, xk_ref, w0b, w0c, b0, wsc, bsc,
                 s0_ref, q0_ref, ssc_ref, qsc_ref):
    hs = _h0_list(xc_ref, xk_ref, w0b, w0c, b0)
    _sumsq_accum(s0_ref, q0_ref, hs)
    scl = _dot(wsc[...], xc_ref[...]) + bsc[...]
    _accum(ssc_ref, jnp.sum(scl, axis=1, keepdims=True))
    _accum(qsc_ref, jnp.sum(scl * scl, axis=1, keepdims=True))


def _layer(hs, s, q, g, be, w, bias):
    a, c = _bn_const(s, q, g, be, NK_CNT)
    out = []
    for h in hs:
        r = jnp.maximum(a * h + c, 0.0)
        out.append(_dot(w[...], r) + bias[...])
    return out


def _stats1_body(xc_ref, xk_ref, w0b, w0c, b0, s0, q0, g0, be0, w1, b1,
                 s1_ref, q1_ref):
    hs = _h0_list(xc_ref, xk_ref, w0b, w0c, b0)
    hs = _layer(hs, s0, q0, g0, be0, w1, b1)
    _sumsq_accum(s1_ref, q1_ref, hs)


def _stats2_body(xc_ref, xk_ref, w0b, w0c, b0, s0, q0, g0, be0, w1, b1,
                 s1, q1, g1, be1, w2, b2, s2_ref, q2_ref):
    hs = _h0_list(xc_ref, xk_ref, w0b, w0c, b0)
    hs = _layer(hs, s0, q0, g0, be0, w1, b1)
    hs = _layer(hs, s1, q1, g1, be1, w2, b2)
    _sumsq_accum(s2_ref, q2_ref, hs)


def _final_body(xc_ref, xk_ref, w0b, w0c, b0, s0, q0, g0, be0, w1, b1,
                s1, q1, g1, be1, w2, b2, s2, q2, g2, be2,
                wsc, bsc, gsc, besc, ssc, qsc, out_ref):
    hs = _h0_list(xc_ref, xk_ref, w0b, w0c, b0)
    hs = _layer(hs, s0, q0, g0, be0, w1, b1)
    hs = _layer(hs, s1, q1, g1, be1, w2, b2)
    a2, c2 = _bn_const(s2, q2, g2, be2, NK_CNT)
    acc = jnp.zeros((C, RM), jnp.float32)
    for h in hs:
        acc += jnp.maximum(a2 * h + c2, 0.0)
    aggr = acc * jnp.float32(1.0 / K)
    scl = _dot(wsc[...], xc_ref[...]) + bsc[...]
    asc, csc = _bn_const(ssc, qsc, gsc, besc, NSC_CNT)
    out_ref[...] = jnp.maximum(aggr + asc * scl + csc, 0.0)


def _cspec(shape):
    return pl.BlockSpec(shape, lambda i: tuple(0 for _ in shape))


def kernel(x, W_sc, b_sc, g_sc, be_sc, W0, b0, g0, be0,
           W1, b1, g1, be1, W2, b2, g2, be2):
    idx = _knn_call(x)
    xk3 = _gather_call(x, idx)
    xt = jnp.transpose(x.reshape(B * N, D))

    w0b = W0[:, D:]
    w0c = W0[:, :D] - W0[:, D:]
    col = lambda v: v.reshape(C, 1)
    b0c, b1c, b2c, bscc = col(b0), col(b1), col(b2), col(b_sc)
    g0c, be0c = col(g0), col(be0)
    g1c, be1c = col(g1), col(be1)
    g2c, be2c = col(g2), col(be2)
    gscc, bescc = col(g_sc), col(be_sc)

    grid = (B * N // RM,)
    row_specs = [
        pl.BlockSpec((D, RM), lambda i: (0, i)),
        pl.BlockSpec((K * D, RM), lambda i: (0, i)),
    ]
    wdc = _cspec((C, D))
    wcc = _cspec((C, C))
    vsp = _cspec((C, 1))
    acc_spec = pl.BlockSpec((C, 1), lambda i: (0, 0))
    acc_shape = jax.ShapeDtypeStruct((C, 1), jnp.float32)

    s0, q0, ssc, qsc = pl.pallas_call(
        _stats0_body,
        grid=grid,
        in_specs=row_specs + [wdc, wdc, vsp, wdc, vsp],
        out_specs=[acc_spec] * 4,
        out_shape=[acc_shape] * 4,
    )(xt, xk3, w0b, w0c, b0c, W_sc, bscc)

    s1, q1 = pl.pallas_call(
        _stats1_body,
        grid=grid,
        in_specs=row_specs + [wdc, wdc, vsp, vsp, vsp, vsp, vsp, wcc, vsp],
        out_specs=[acc_spec] * 2,
        out_shape=[acc_shape] * 2,
    )(xt, xk3, w0b, w0c, b0c, s0, q0, g0c, be0c, W1, b1c)

    s2, q2 = pl.pallas_call(
        _stats2_body,
        grid=grid,
        in_specs=row_specs + [wdc, wdc, vsp, vsp, vsp, vsp, vsp, wcc, vsp,
                              vsp, vsp, vsp, vsp, wcc, vsp],
        out_specs=[acc_spec] * 2,
        out_shape=[acc_shape] * 2,
    )(xt, xk3, w0b, w0c, b0c, s0, q0, g0c, be0c, W1, b1c,
      s1, q1, g1c, be1c, W2, b2c)

    out_t = pl.pallas_call(
        _final_body,
        grid=grid,
        in_specs=row_specs + [wdc, wdc, vsp, vsp, vsp, vsp, vsp, wcc, vsp,
                              vsp, vsp, vsp, vsp, wcc, vsp, vsp, vsp, vsp,
                              vsp, wdc, vsp, vsp, vsp, vsp, vsp],
        out_specs=pl.BlockSpec((C, RM), lambda i: (0, i)),
        out_shape=jax.ShapeDtypeStruct((C, B * N), jnp.float32),
    )(xt, xk3, w0b, w0c, b0c, s0, q0, g0c, be0c, W1, b1c,
      s1, q1, g1c, be1c, W2, b2c, s2, q2, g2c, be2c,
      W_sc, bscc, gscc, bescc, ssc, qsc)

    return jnp.transpose(out_t).reshape(B, N, C)


# speedup vs baseline: 17.2148x; 1.4933x over previous
import functools

import jax
import jax.numpy as jnp
from jax import lax
from jax.experimental import pallas as pl
from jax.experimental.pallas import tpu as pltpu
from jax.experimental.pallas import tpu_sc as plsc

B, N, D, K, C = 16, 1024, 16, 20, 64
EPS = 1e-5
NKTOT = B * N * K
NK_CNT = float(B * N * K)
NSC_CNT = float(B * N)
RN = 256
RM = 512
LANES = 128
NROWS = NKTOT // LANES
NW = 32
CHUNKS_PER_W = NROWS // NW
PTS_PER_ROW = LANES // D


def _knn_body(x_ref, xt_ref, idx_ref):
    b = pl.program_id(0)
    xj_p = x_ref[0, :, 0:1]
    xj_e = x_ref[0, :, 1:2]
    xi_p = xt_ref[0:1, :]
    xi_e = xt_ref[1:2, :]
    dx = xj_p - xi_p
    dy = xj_e - xi_e
    s = dx * dx + dy * dy
    jota = lax.broadcasted_iota(jnp.int32, (N, RN), 0)
    key = (lax.bitcast_convert_type(s, jnp.int32) & ~(N - 1)) | jota
    big = jnp.int32(0x7FFFFFFF)
    for k in range(K):
        mk = jnp.min(key, axis=0, keepdims=True)
        idx_ref[k:k + 1, :] = (mk & (N - 1)) + b * N
        if k + 1 < K:
            key = jnp.where(key == mk, big, key)


def _knn_call(x, xt):
    return pl.pallas_call(
        _knn_body,
        grid=(B, N // RN),
        in_specs=[
            pl.BlockSpec((1, N, D), lambda b, r: (b, 0, 0)),
            pl.BlockSpec((8, RN), lambda b, r: (0, b * (N // RN) + r)),
        ],
        out_specs=pl.BlockSpec((K, RN), lambda b, r: (0, b * (N // RN) + r)),
        out_shape=jax.ShapeDtypeStruct((K, B * N), jnp.int32),
    )(x, xt)


CH = 512


def _gather_body(xflat_hbm, idx_hbm, out_hbm,
                 x_tile, idx_v, p0, p1, sem0, sem1):
    wid = lax.axis_index("s") * 2 + lax.axis_index("c")
    b = wid // 2
    col0 = wid * CH
    pltpu.sync_copy(xflat_hbm.at[pl.ds(b * N * D, N * D)], x_tile)

    def gather_one(t, pk):
        pltpu.sync_copy(idx_hbm.at[t, pl.ds(col0, CH)], idx_v)
        for c in range(CH // 16):
            iv = idx_v[pl.ds(16 * c, 16)] & (N - 1)
            fbase = iv << 4
            for d in range(D):
                pk[d, pl.ds(16 * c, 16)] = plsc.load_gather(
                    x_tile, [fbase + d])

    def pair(u, carry):
        for h, (pk, sem) in enumerate(((p0, sem0), (p1, sem1))):
            t = u * 2 + h

            @pl.when(u >= 1)
            def _():
                pltpu.make_async_copy(
                    pk, out_hbm.at[pl.ds(0, D), pl.ds(col0, CH)], sem).wait()

            gather_one(t, pk)
            pltpu.async_copy(
                pk, out_hbm.at[pl.ds(t * D, D), pl.ds(col0, CH)], sem)
        return carry

    lax.fori_loop(0, K // 2, pair, 0)
    pltpu.make_async_copy(
        p0, out_hbm.at[pl.ds(0, D), pl.ds(col0, CH)], sem0).wait()
    pltpu.make_async_copy(
        p1, out_hbm.at[pl.ds(0, D), pl.ds(col0, CH)], sem1).wait()


def _gather_call(x, idx):
    xflat = x.reshape(B * N * D)
    mesh = plsc.VectorSubcoreMesh(core_axis_name="c", subcore_axis_name="s")
    fn = functools.partial(
        pl.kernel,
        mesh=mesh,
        compiler_params=pltpu.CompilerParams(needs_layout_passes=False),
        out_type=jax.ShapeDtypeStruct((K * D, B * N), jnp.float32),
        scratch_types=[
            pltpu.VMEM((N * D,), jnp.float32),
            pltpu.VMEM((CH,), jnp.int32),
            pltpu.VMEM((D, CH), jnp.float32),
            pltpu.VMEM((D, CH), jnp.float32),
            pltpu.SemaphoreType.DMA,
            pltpu.SemaphoreType.DMA,
        ],
    )(_gather_body)
    return fn(xflat, idx)


def _bn_const(s_ref, q_ref, g_ref, be_ref, cnt):
    mean = s_ref[...] / cnt
    var = q_ref[...] / cnt - mean * mean
    a = g_ref[...] * lax.rsqrt(var + EPS)
    c = be_ref[...] - a * mean
    return a, c


def _dot(a, b):
    return jnp.dot(a.astype(jnp.bfloat16), b.astype(jnp.bfloat16),
                   preferred_element_type=jnp.float32)


def _h0_list(xc_ref, xk_ref, w0b_ref, w0c_ref, b0_ref):
    hc = _dot(w0c_ref[...], xc_ref[...]) + b0_ref[...]
    out = []
    for k in range(K):
        xkk = xk_ref[k * D:(k + 1) * D, :]
        out.append(_dot(w0b_ref[...], xkk) + hc)
    return out


def _accum(ref, val):
    @pl.when(pl.program_id(0) == 0)
    def _():
        ref[...] = jnp.zeros_like(ref)
    ref[...] += val


def _sumsq_accum(s_ref, q_ref, hs):
    s = jnp.zeros((C, 1), jnp.float32)
    q = jnp.zeros((C, 1), jnp.float32)
    for h in hs:
        s += jnp.sum(h, axis=1, keepdims=True)
        q += jnp.sum(h * h, axis=1, keepdims=True)
    _accum(s_ref, s)
    _accum(q_ref, q)


def _stats0_body(xc_ref, xk_ref, w0b, w0c, b0, wsc, bsc,
                 s0_ref, q0_ref, ssc_ref, qsc_ref):
    hs = _h0_list(xc_ref, xk_ref, w0b, w0c, b0)
    _sumsq_accum(s0_ref, q0_ref, hs)
    scl = _dot(wsc[...], xc_ref[...]) + bsc[...]
    _accum(ssc_ref, jnp.sum(scl, axis=1, keepdims=True))
    _accum(qsc_ref, jnp.sum(scl * scl, axis=1, keepdims=True))


def _layer(hs, s, q, g, be, w, bias):
    a, c = _bn_const(s, q, g, be, NK_CNT)
    out = []
    for h in hs:
        r = jnp.maximum(a * h + c, 0.0)
        out.append(_dot(w[...], r) + bias[...])
    return out


def _stats1_body(xc_ref, xk_ref, w0b, w0c, b0, s0, q0, g0, be0, w1, b1,
                 s1_ref, q1_ref):
    hs = _h0_list(xc_ref, xk_ref, w0b, w0c, b0)
    hs = _layer(hs, s0, q0, g0, be0, w1, b1)
    _sumsq_accum(s1_ref, q1_ref, hs)


def _stats2_body(xc_ref, xk_ref, w0b, w0c, b0, s0, q0, g0, be0, w1, b1,
                 s1, q1, g1, be1, w2, b2, s2_ref, q2_ref):
    hs = _h0_list(xc_ref, xk_ref, w0b, w0c, b0)
    hs = _layer(hs, s0, q0, g0, be0, w1, b1)
    hs = _layer(hs, s1, q1, g1, be1, w2, b2)
    _sumsq_accum(s2_ref, q2_ref, hs)


def _final_body(xc_ref, xk_ref, w0b, w0c, b0, s0, q0, g0, be0, w1, b1,
                s1, q1, g1, be1, w2, b2, s2, q2, g2, be2,
                wsc, bsc, gsc, besc, ssc, qsc, out_ref):
    hs = _h0_list(xc_ref, xk_ref, w0b, w0c, b0)
    hs = _layer(hs, s0, q0, g0, be0, w1, b1)
    hs = _layer(hs, s1, q1, g1, be1, w2, b2)
    a2, c2 = _bn_const(s2, q2, g2, be2, NK_CNT)
    acc = jnp.zeros((C, RM), jnp.float32)
    for h in hs:
        acc += jnp.maximum(a2 * h + c2, 0.0)
    aggr = acc * jnp.float32(1.0 / K)
    scl = _dot(wsc[...], xc_ref[...]) + bsc[...]
    asc, csc = _bn_const(ssc, qsc, gsc, besc, NSC_CNT)
    out_ref[...] = jnp.maximum(aggr + asc * scl + csc, 0.0)


def _cspec(shape):
    return pl.BlockSpec(shape, lambda i: tuple(0 for _ in shape))


def kernel(x, W_sc, b_sc, g_sc, be_sc, W0, b0, g0, be0,
           W1, b1, g1, be1, W2, b2, g2, be2):
    xt = jnp.transpose(x.reshape(B * N, D))
    idx = _knn_call(x, xt)
    xk3 = _gather_call(x, idx)

    w0b = W0[:, D:]
    w0c = W0[:, :D] - W0[:, D:]
    col = lambda v: v.reshape(C, 1)
    b0c, b1c, b2c, bscc = col(b0), col(b1), col(b2), col(b_sc)
    g0c, be0c = col(g0), col(be0)
    g1c, be1c = col(g1), col(be1)
    g2c, be2c = col(g2), col(be2)
    gscc, bescc = col(g_sc), col(be_sc)

    grid = (B * N // RM,)
    row_specs = [
        pl.BlockSpec((D, RM), lambda i: (0, i)),
        pl.BlockSpec((K * D, RM), lambda i: (0, i)),
    ]
    wdc = _cspec((C, D))
    wcc = _cspec((C, C))
    vsp = _cspec((C, 1))
    acc_spec = pl.BlockSpec((C, 1), lambda i: (0, 0))
    acc_shape = jax.ShapeDtypeStruct((C, 1), jnp.float32)

    s0, q0, ssc, qsc = pl.pallas_call(
        _stats0_body,
        grid=grid,
        in_specs=row_specs + [wdc, wdc, vsp, wdc, vsp],
        out_specs=[acc_spec] * 4,
        out_shape=[acc_shape] * 4,
    )(xt, xk3, w0b, w0c, b0c, W_sc, bscc)

    s1, q1 = pl.pallas_call(
        _stats1_body,
        grid=grid,
        in_specs=row_specs + [wdc, wdc, vsp, vsp, vsp, vsp, vsp, wcc, vsp],
        out_specs=[acc_spec] * 2,
        out_shape=[acc_shape] * 2,
    )(xt, xk3, w0b, w0c, b0c, s0, q0, g0c, be0c, W1, b1c)

    s2, q2 = pl.pallas_call(
        _stats2_body,
        grid=grid,
        in_specs=row_specs + [wdc, wdc, vsp, vsp, vsp, vsp, vsp, wcc, vsp,
                              vsp, vsp, vsp, vsp, wcc, vsp],
        out_specs=[acc_spec] * 2,
        out_shape=[acc_shape] * 2,
    )(xt, xk3, w0b, w0c, b0c, s0, q0, g0c, be0c, W1, b1c,
      s1, q1, g1c, be1c, W2, b2c)

    out_t = pl.pallas_call(
        _final_body,
        grid=grid,
        in_specs=row_specs + [wdc, wdc, vsp, vsp, vsp, vsp, vsp, wcc, vsp,
                              vsp, vsp, vsp, vsp, wcc, vsp, vsp, vsp, vsp,
                              vsp, wdc, vsp, vsp, vsp, vsp, vsp],
        out_specs=pl.BlockSpec((C, RM), lambda i: (0, i)),
        out_shape=jax.ShapeDtypeStruct((C, B * N), jnp.float32),
    )(xt, xk3, w0b, w0c, b0c, s0, q0, g0c, be0c, W1, b1c,
      s1, q1, g1c, be1c, W2, b2c, s2, q2, g2c, be2c,
      W_sc, bscc, gscc, bescc, ssc, qsc)

    return jnp.transpose(out_t).reshape(B, N, C)
